# baseline (speedup 1.0000x reference)
import jax
import jax.numpy as jnp
from jax import lax
from jax.experimental import pallas as pl
from jax.experimental.pallas import tpu as pltpu
from jax.experimental.pallas import tpu_sc as plsc

NC, NS, L = 2, 16, 16
NW = NC * NS
NROW = 2048
NVIS = 512
D = 4096
RPW = NROW // NW
C = 4
NCHUNK = RPW // C
NBUF = 3


def _body(vm_hbm, red_hbm, v2r_hbm, out_hbm,
          idx_v, red_buf, vm_buf, sems):
    wid = lax.axis_index("c") * NS + lax.axis_index("s")
    base = wid * RPW
    p1 = base // (NROW // 2)
    p2 = (base // NVIS) % 2
    vis0 = base % NVIS

    pltpu.sync_copy(v2r_hbm.at[pl.ds(vis0, RPW)], idx_v)
    idx_vecs = [idx_v[pl.ds(k * L, L)] for k in range(RPW // L)]

    def start_loads(g):
        b = g % NBUF
        ds = []
        for r in range(C):
            t = g * C + r
            j = idx_vecs[t // L][t % L]
            ds.append(pltpu.async_copy(
                red_hbm.at[p1, p2, j], red_buf.at[b, r], sems.at[0, b]))
            ds.append(pltpu.async_copy(
                vm_hbm.at[p1, p2, vis0 + t], vm_buf.at[b, r],
                sems.at[1, b]))
        return ds

    def start_out(g):
        b = g % NBUF
        return [pltpu.async_copy(
            vm_buf.at[b, r], out_hbm.at[p1, p2, vis0 + g * C + r],
            sems.at[2, b])
            for r in range(C)]

    loads = [None] * NCHUNK
    outs = [None] * NCHUNK
    for g in range(min(NBUF, NCHUNK)):
        loads[g] = start_loads(g)
    for g in range(NCHUNK):
        for d in loads[g]:
            d.wait()
        b = g % NBUF
        for r in range(C):
            @plsc.parallel_loop(0, D // L, unroll=8)
            def _(j):
                sl = pl.ds(j * L, L)
                plsc.addupdate(vm_buf.at[b, r, sl], red_buf[b, r, sl])
        outs[g] = start_out(g)
        if g >= 1 and g - 1 + NBUF < NCHUNK:
            for d in outs[g - 1]:
                d.wait()
            loads[g - 1 + NBUF] = start_loads(g - 1 + NBUF)
    for g in range(max(0, NCHUNK - NBUF), NCHUNK):
        for d in outs[g]:
            d.wait()


def kernel(V_m, red, vis2red):
    vm4 = V_m.reshape(2, 2, NVIS, D)
    red4 = red.reshape(2, 2, 64, D)
    mesh = plsc.VectorSubcoreMesh(core_axis_name="c", subcore_axis_name="s",
                                  num_cores=NC, num_subcores=NS)
    out = pl.kernel(
        _body,
        out_type=jax.ShapeDtypeStruct((2, 2, NVIS, D), jnp.float32),
        mesh=mesh,
        scratch_types=[
            pltpu.VMEM((RPW,), jnp.int32),
            pltpu.VMEM((NBUF, C, D), jnp.float32),
            pltpu.VMEM((NBUF, C, D), jnp.float32),
            pltpu.SemaphoreType.DMA((3, NBUF)),
        ],
    )(vm4, red4, vis2red)
    return out.reshape(V_m.shape)
